# baseline (speedup 1.0000x reference)
import functools

import jax
import jax.numpy as jnp
from jax.experimental import pallas as pl
from jax.experimental.pallas import tpu as pltpu

_LANES = 128


def _group_reduce(r, op, groups, width):
    acc = r[:, :width]
    for g in range(1, groups):
        acc = op(acc, r[:, g * width:(g + 1) * width])
    return acc


def _mixmil_kernel(qmu_ref, qls_ref, eps_ref, x_ref, out_ref, wblk_ref, acc_ref,
                   *, n_bags):
    n = pl.program_id(0)
    s2 = eps_ref.shape[1] * 2
    groups = _LANES // s2

    beta = qmu_ref[...] + jnp.exp(qls_ref[...]) * eps_ref[...]
    q = beta.shape[0] // 2
    beta_u = beta[:q]
    beta_z = beta[q:]
    b = jnp.sqrt(jnp.mean(beta_z * beta_z, axis=0, keepdims=True))
    eta = beta_z / b

    @pl.when(n == 0)
    def _build_wblock():
        w = jnp.concatenate([beta_u, eta], axis=1)
        wt = jnp.tile(w, (groups, groups))
        row_blk = jax.lax.broadcasted_iota(jnp.int32, wt.shape, 0) // q
        lane_blk = jax.lax.broadcasted_iota(jnp.int32, wt.shape, 1) // s2
        wblk_ref[...] = jnp.where(row_blk == lane_blk, wt, 0.0)

    x = x_ref[0]
    dn = (((1,), (0,)), ((), ()))
    y = jax.lax.dot_general(x, wblk_ref[...], dn,
                            preferred_element_type=jnp.float32)

    colmax = jnp.max(y, axis=0, keepdims=True)
    m16 = _group_reduce(colmax, jnp.maximum, groups, s2)
    m = jnp.concatenate([m16] * groups, axis=1)
    e = jnp.exp(y - m)
    zroll = jnp.roll(y, -s2 // 2, axis=1)
    denom = _group_reduce(jnp.sum(e, axis=0, keepdims=True), jnp.add,
                          groups, s2)
    num = _group_reduce(jnp.sum(e * zroll, axis=0, keepdims=True), jnp.add,
                        groups, s2)
    acc_ref[pl.ds(n, 1), :] = num / denom

    @pl.when(n == n_bags - 1)
    def _finalize():
        xm = acc_ref[...]
        mean = jnp.mean(xm, axis=0, keepdims=True)
        d = xm - mean
        var = jnp.sum(d * d, axis=0, keepdims=True) / (n_bags - 1)
        res = b * d[:, :s2 // 2] / jnp.sqrt(var[:, :s2 // 2])
        out_ref[...] = res


def kernel(Xs, q_mu, q_log_sigma, eps):
    n_bags, i_inst, q_dim = Xs.shape
    two_q, p_dim, s_dim = eps.shape
    ps = p_dim * s_dim
    groups = _LANES // (2 * ps)
    eps2 = eps.reshape(two_q, ps)
    xs_v = Xs.reshape(n_bags, i_inst // groups, groups * q_dim)

    out = pl.pallas_call(
        functools.partial(_mixmil_kernel, n_bags=n_bags),
        grid=(n_bags,),
        in_specs=[
            pl.BlockSpec((two_q, p_dim), lambda n: (0, 0)),
            pl.BlockSpec((two_q, p_dim), lambda n: (0, 0)),
            pl.BlockSpec((two_q, ps), lambda n: (0, 0)),
            pl.BlockSpec((1, i_inst // groups, groups * q_dim),
                         lambda n: (n, 0, 0)),
        ],
        out_specs=pl.BlockSpec((n_bags, ps), lambda n: (0, 0)),
        out_shape=jax.ShapeDtypeStruct((n_bags, ps), jnp.float32),
        scratch_shapes=[
            pltpu.VMEM((groups * q_dim, _LANES), jnp.float32),
            pltpu.VMEM((n_bags, 2 * ps), jnp.float32),
        ],
    )(q_mu, q_log_sigma, eps2, xs_v)
    return out.reshape(n_bags, p_dim, s_dim)
